# baseline (speedup 1.0000x reference)
import jax
import jax.numpy as jnp
from jax.experimental import pallas as pl

_B, _N, _C = 16, 4096, 3
_ROWS_PER_BATCH = 32
_LANES = _N * _C // _ROWS_PER_BATCH


def _voxel_body(x_ref, o_ref):
    x = x_ref[...]
    xb = x.reshape(_B, _ROWS_PER_BATCH, _LANES)
    part_mn = jnp.min(xb, axis=1)
    part_mx = jnp.max(xb, axis=1)

    lane = jax.lax.broadcasted_iota(jnp.int32, (_B, _LANES), 1) % _C
    m0 = lane == 0
    m1 = lane == 1

    inf = jnp.float32(jnp.inf)
    mn0 = jnp.min(jnp.where(m0, part_mn, inf), axis=1, keepdims=True)
    mn1 = jnp.min(jnp.where(m1, part_mn, inf), axis=1, keepdims=True)
    mn2 = jnp.min(jnp.where(lane == 2, part_mn, inf), axis=1, keepdims=True)
    mx0 = jnp.max(jnp.where(m0, part_mx, -inf), axis=1, keepdims=True)
    mx1 = jnp.max(jnp.where(m1, part_mx, -inf), axis=1, keepdims=True)
    mx2 = jnp.max(jnp.where(lane == 2, part_mx, -inf), axis=1, keepdims=True)

    mn_row = jnp.where(m0, mn0, jnp.where(m1, mn1, mn2))
    mx_row = jnp.where(m0, mx0, jnp.where(m1, mx1, mx2))
    bw_row = (mx_row - mn_row) / 40.0

    mn_full = jnp.broadcast_to(mn_row[:, None, :], xb.shape).reshape(x.shape)
    bw_full = jnp.broadcast_to(bw_row[:, None, :], xb.shape).reshape(x.shape)
    o_ref[...] = jnp.floor((x - mn_full) / bw_full)


def kernel(point_cloud):
    b, n, c = point_cloud.shape
    flat = point_cloud.reshape(b * _ROWS_PER_BATCH, _LANES)
    out = pl.pallas_call(
        _voxel_body,
        out_shape=jax.ShapeDtypeStruct(flat.shape, jnp.float32),
    )(flat)
    return out.reshape(b, n, c)
